# Initial kernel scaffold; baseline (speedup 1.0000x reference)
import jax
import jax.numpy as jnp
from jax.experimental import pallas as pl


def kernel(morganSMILES, table, W, b):
    raise NotImplementedError("write your pallas kernel here")


# speedup vs baseline: 8.3839x; 8.3839x over previous
import functools

import jax
import jax.numpy as jnp
from jax import lax
from jax.experimental import pallas as pl
from jax.experimental.pallas import tpu as pltpu
from jax.experimental.pallas import tpu_sc as plsc

VOCAB = 100000
EMBED_DIM = 300
D_MODEL = 128

_ROW_TILE = 4000

_NC = 2
_NS = 16
_NW = _NC * _NS
_CHUNK = 128


def _proj_body(t_ref, w_ref, b_ref, o_ref):
    o_ref[...] = (
        jnp.dot(t_ref[...], w_ref[...], preferred_element_type=jnp.float32)
        + b_ref[...]
    )


def _project_table(table, W, b):
    return pl.pallas_call(
        _proj_body,
        grid=(VOCAB // _ROW_TILE,),
        in_specs=[
            pl.BlockSpec((_ROW_TILE, EMBED_DIM), lambda i: (i, 0)),
            pl.BlockSpec((EMBED_DIM, D_MODEL), lambda i: (0, 0)),
            pl.BlockSpec((1, D_MODEL), lambda i: (0, 0)),
        ],
        out_specs=pl.BlockSpec((_ROW_TILE, D_MODEL), lambda i: (i, 0)),
        out_shape=jax.ShapeDtypeStruct((VOCAB, D_MODEL), jnp.float32),
    )(table, W, b.reshape(1, D_MODEL))


def _make_gather(n_chunks):
    total = _NW * n_chunks * _CHUNK
    mesh = plsc.VectorSubcoreMesh(core_axis_name="c", subcore_axis_name="s")

    @functools.partial(
        pl.kernel,
        mesh=mesh,
        out_type=jax.ShapeDtypeStruct((total, D_MODEL), jnp.float32),
        scratch_types=[
            pltpu.VMEM((n_chunks, _CHUNK), jnp.int32),
            pltpu.VMEM((_CHUNK, D_MODEL), jnp.float32),
            pltpu.SemaphoreType.DMA,
        ],
    )
    def gather_k(idx_hbm, p_hbm, out_hbm, idx_v, buf, sem):
        wid = lax.axis_index("s") * _NC + lax.axis_index("c")
        pltpu.sync_copy(idx_hbm.at[wid], idx_v)
        base = wid * (n_chunks * _CHUNK)

        def step(j, carry):
            pltpu.async_copy(p_hbm.at[idx_v.at[j]], buf, sem).wait()
            pltpu.sync_copy(buf, out_hbm.at[pl.ds(base + j * _CHUNK, _CHUNK)])
            return carry

        lax.fori_loop(0, n_chunks, step, 0)

    return gather_k


def kernel(morganSMILES, table, W, b):
    B, L = morganSMILES.shape
    total = B * L
    n_chunks = total // (_NW * _CHUNK)
    P = _project_table(table, W, b)
    idx = morganSMILES.reshape(_NW, n_chunks, _CHUNK).astype(jnp.int32)
    out = _make_gather(n_chunks)(idx, P)
    return out.reshape(B, L, D_MODEL)


# speedup vs baseline: 13.6327x; 1.6261x over previous
import functools

import jax
import jax.numpy as jnp
from jax import lax
from jax.experimental import pallas as pl
from jax.experimental.pallas import tpu as pltpu
from jax.experimental.pallas import tpu_sc as plsc

VOCAB = 100000
EMBED_DIM = 300
D_MODEL = 128

_ROW_TILE = 4000

_NC = 2
_NS = 16
_NW = _NC * _NS
_CHUNK = 128


def _proj_body(t_ref, w_ref, b_ref, o_ref):
    o_ref[...] = (
        jnp.dot(t_ref[...], w_ref[...], preferred_element_type=jnp.float32)
        + b_ref[...]
    )


def _project_table(table, W, b):
    return pl.pallas_call(
        _proj_body,
        grid=(VOCAB // _ROW_TILE,),
        in_specs=[
            pl.BlockSpec((_ROW_TILE, EMBED_DIM), lambda i: (i, 0)),
            pl.BlockSpec((EMBED_DIM, D_MODEL), lambda i: (0, 0)),
            pl.BlockSpec((1, D_MODEL), lambda i: (0, 0)),
        ],
        out_specs=pl.BlockSpec((_ROW_TILE, D_MODEL), lambda i: (i, 0)),
        out_shape=jax.ShapeDtypeStruct((VOCAB, D_MODEL), jnp.float32),
    )(table, W, b.reshape(1, D_MODEL))


def _make_gather(n_chunks):
    total = _NW * n_chunks * _CHUNK
    mesh = plsc.VectorSubcoreMesh(core_axis_name="c", subcore_axis_name="s")

    @functools.partial(
        pl.kernel,
        mesh=mesh,
        out_type=jax.ShapeDtypeStruct((total, D_MODEL), jnp.float32),
        scratch_types=[
            pltpu.VMEM((n_chunks, _CHUNK), jnp.int32),
            pltpu.VMEM((_CHUNK, D_MODEL), jnp.float32),
            pltpu.SemaphoreType.DMA,
        ],
    )
    def gather_k(idx_hbm, p_hbm, out_hbm, idx_v, buf, sem):
        wid = lax.axis_index("s") * _NC + lax.axis_index("c")
        pltpu.sync_copy(idx_hbm.at[wid], idx_v)
        base = wid * (n_chunks * _CHUNK)

        def step(j, carry):
            pltpu.async_copy(p_hbm.at[idx_v.at[j]], buf, sem).wait()
            pltpu.sync_copy(buf, out_hbm.at[pl.ds(base + j * _CHUNK, _CHUNK)])
            return carry

        lax.fori_loop(0, n_chunks, step, 0)

    return gather_k


def kernel(morganSMILES, table, W, b):
    B, L = morganSMILES.shape
    total = B * L
    n_chunks = total // (_NW * _CHUNK)
    P = _project_table(table, W, b)
    idx = morganSMILES.T.reshape(_NW, n_chunks, _CHUNK).astype(jnp.int32)
    out = _make_gather(n_chunks)(idx, P)
    return out.reshape(L, B, D_MODEL).transpose(1, 0, 2)


# speedup vs baseline: 15.2433x; 1.1181x over previous
import functools

import jax
import jax.numpy as jnp
from jax import lax
from jax.experimental import pallas as pl
from jax.experimental.pallas import tpu as pltpu
from jax.experimental.pallas import tpu_sc as plsc

VOCAB = 100000
EMBED_DIM = 300
D_MODEL = 128

_ROW_TILE = 4000

_NC = 2
_NS = 16
_NW = _NC * _NS
_CHUNK = 128


def _proj_body(t_ref, w_ref, b_ref, o_ref):
    o_ref[...] = (
        jnp.dot(t_ref[...], w_ref[...], preferred_element_type=jnp.float32)
        + b_ref[...]
    )


def _project_table(table, W, b):
    return pl.pallas_call(
        _proj_body,
        grid=(VOCAB // _ROW_TILE,),
        in_specs=[
            pl.BlockSpec((_ROW_TILE, EMBED_DIM), lambda i: (i, 0)),
            pl.BlockSpec((EMBED_DIM, D_MODEL), lambda i: (0, 0)),
            pl.BlockSpec((1, D_MODEL), lambda i: (0, 0)),
        ],
        out_specs=pl.BlockSpec((_ROW_TILE, D_MODEL), lambda i: (i, 0)),
        out_shape=jax.ShapeDtypeStruct((VOCAB, D_MODEL), jnp.float32),
    )(table, W, b.reshape(1, D_MODEL))


_NBUF = 5


def _make_gather(n_chunks):
    total = _NW * n_chunks * _CHUNK
    ngroups = n_chunks // _NBUF
    assert ngroups * _NBUF == n_chunks
    mesh = plsc.VectorSubcoreMesh(core_axis_name="c", subcore_axis_name="s")
    scratch = [pltpu.VMEM((n_chunks, _CHUNK), jnp.int32)]
    scratch += [pltpu.VMEM((_CHUNK, D_MODEL), jnp.float32) for _ in range(_NBUF)]
    scratch += [pltpu.SemaphoreType.DMA for _ in range(2 * _NBUF)]

    @functools.partial(
        pl.kernel,
        mesh=mesh,
        out_type=jax.ShapeDtypeStruct((total, D_MODEL), jnp.float32),
        scratch_types=scratch,
    )
    def gather_k(idx_hbm, p_hbm, out_hbm, idx_v, *bufs_sems):
        bufs = bufs_sems[:_NBUF]
        gsems = bufs_sems[_NBUF : 2 * _NBUF]
        wsems = bufs_sems[2 * _NBUF :]
        wid = lax.axis_index("s") * _NC + lax.axis_index("c")
        pltpu.sync_copy(idx_hbm.at[wid], idx_v)
        base = wid * (n_chunks * _CHUNK)

        def gath(j, bi):
            return pltpu.make_async_copy(p_hbm.at[idx_v.at[j]], bufs[bi], gsems[bi])

        def wb(j, bi):
            return pltpu.make_async_copy(
                bufs[bi], out_hbm.at[pl.ds(base + j * _CHUNK, _CHUNK)], wsems[bi]
            )

        for bi in range(_NBUF):
            gath(bi, bi).start()

        def group(g, carry):
            j0 = g * _NBUF
            for bi in range(_NBUF):
                gath(j0 + bi, bi).wait()
                wb(j0 + bi, bi).start()

            @pl.when(g + 1 < ngroups)
            def _():
                for bi in range(_NBUF):
                    wb(j0 + bi, bi).wait()
                    gath(j0 + _NBUF + bi, bi).start()

            return carry

        lax.fori_loop(0, ngroups, group, 0)
        for bi in range(_NBUF):
            wb((ngroups - 1) * _NBUF + bi, bi).wait()

    return gather_k


def kernel(morganSMILES, table, W, b):
    B, L = morganSMILES.shape
    total = B * L
    n_chunks = total // (_NW * _CHUNK)
    P = _project_table(table, W, b)
    idx = morganSMILES.T.reshape(_NW, n_chunks, _CHUNK).astype(jnp.int32)
    out = _make_gather(n_chunks)(idx, P)
    return out.reshape(L, B, D_MODEL).transpose(1, 0, 2)


# speedup vs baseline: 27.3533x; 1.7944x over previous
import functools

import jax
import jax.numpy as jnp
from jax import lax
from jax.experimental import pallas as pl
from jax.experimental.pallas import tpu as pltpu
from jax.experimental.pallas import tpu_sc as plsc

VOCAB = 100000
EMBED_DIM = 300
D_MODEL = 128

_ROW_TILE = 4096

_NC = 2
_NS = 16
_NW = _NC * _NS
_CHUNK = 128


def _proj_body(tT_ref, w_ref, b_ref, o_ref):
    o_ref[...] = (
        lax.dot_general(
            tT_ref[...],
            w_ref[...],
            dimension_numbers=(((0,), (0,)), ((), ())),
            preferred_element_type=jnp.float32,
        )
        + b_ref[...]
    )


def _project_table(tableT, W, b):
    grid = (VOCAB + _ROW_TILE - 1) // _ROW_TILE
    return pl.pallas_call(
        _proj_body,
        grid=(grid,),
        in_specs=[
            pl.BlockSpec((EMBED_DIM, _ROW_TILE), lambda i: (0, i)),
            pl.BlockSpec((EMBED_DIM, D_MODEL), lambda i: (0, 0)),
            pl.BlockSpec((1, D_MODEL), lambda i: (0, 0)),
        ],
        out_specs=pl.BlockSpec((_ROW_TILE, D_MODEL), lambda i: (i, 0)),
        out_shape=jax.ShapeDtypeStruct((VOCAB, D_MODEL), jnp.float32),
    )(tableT, W, b.reshape(1, D_MODEL))


_NBUF = 5


def _make_gather(n_chunks):
    total = _NW * n_chunks * _CHUNK
    ngroups = n_chunks // _NBUF
    assert ngroups * _NBUF == n_chunks
    mesh = plsc.VectorSubcoreMesh(core_axis_name="c", subcore_axis_name="s")
    scratch = [pltpu.VMEM((n_chunks, _CHUNK), jnp.int32)]
    scratch += [pltpu.VMEM((_CHUNK, D_MODEL), jnp.float32) for _ in range(_NBUF)]
    scratch += [pltpu.SemaphoreType.DMA for _ in range(2 * _NBUF)]

    @functools.partial(
        pl.kernel,
        mesh=mesh,
        out_type=jax.ShapeDtypeStruct((total, D_MODEL), jnp.float32),
        scratch_types=scratch,
    )
    def gather_k(idx_hbm, p_hbm, out_hbm, idx_v, *bufs_sems):
        bufs = bufs_sems[:_NBUF]
        gsems = bufs_sems[_NBUF : 2 * _NBUF]
        wsems = bufs_sems[2 * _NBUF :]
        wid = lax.axis_index("s") * _NC + lax.axis_index("c")
        pltpu.sync_copy(idx_hbm.at[wid], idx_v)
        base = wid * (n_chunks * _CHUNK)

        def gath(j, bi):
            return pltpu.make_async_copy(p_hbm.at[idx_v.at[j]], bufs[bi], gsems[bi])

        def wb(j, bi):
            return pltpu.make_async_copy(
                bufs[bi], out_hbm.at[pl.ds(base + j * _CHUNK, _CHUNK)], wsems[bi]
            )

        for bi in range(_NBUF):
            gath(bi, bi).start()

        def group(g, carry):
            j0 = g * _NBUF
            for bi in range(_NBUF):
                gath(j0 + bi, bi).wait()
                wb(j0 + bi, bi).start()

            @pl.when(g + 1 < ngroups)
            def _():
                for bi in range(_NBUF):
                    wb(j0 + bi, bi).wait()
                    gath(j0 + _NBUF + bi, bi).start()

            return carry

        lax.fori_loop(0, ngroups, group, 0)
        for bi in range(_NBUF):
            wb((ngroups - 1) * _NBUF + bi, bi).wait()

    return gather_k


def kernel(morganSMILES, table, W, b):
    B, L = morganSMILES.shape
    total = B * L
    n_chunks = total // (_NW * _CHUNK)
    P = _project_table(table.T, W, b)
    idx = morganSMILES.T.reshape(_NW, n_chunks, _CHUNK).astype(jnp.int32)
    out = _make_gather(n_chunks)(idx, P)
    return out.reshape(L, B, D_MODEL).transpose(1, 0, 2)


# speedup vs baseline: 27.9522x; 1.0219x over previous
import functools

import jax
import jax.numpy as jnp
from jax import lax
from jax.experimental import pallas as pl
from jax.experimental.pallas import tpu as pltpu
from jax.experimental.pallas import tpu_sc as plsc

VOCAB = 100000
EMBED_DIM = 300
D_MODEL = 128

_ROW_TILE = 4096

_NC = 2
_NS = 16
_NW = _NC * _NS
_CHUNK = 128


def _proj_body(tT_ref, w_ref, b_ref, o_ref):
    o_ref[...] = (
        lax.dot_general(
            tT_ref[...],
            w_ref[...],
            dimension_numbers=(((0,), (0,)), ((), ())),
            preferred_element_type=jnp.float32,
        )
        + b_ref[...]
    )


def _project_table(tableT, W, b):
    grid = (VOCAB + _ROW_TILE - 1) // _ROW_TILE
    return pl.pallas_call(
        _proj_body,
        grid=(grid,),
        in_specs=[
            pl.BlockSpec((EMBED_DIM, _ROW_TILE), lambda i: (0, i)),
            pl.BlockSpec((EMBED_DIM, D_MODEL), lambda i: (0, 0)),
            pl.BlockSpec((1, D_MODEL), lambda i: (0, 0)),
        ],
        out_specs=pl.BlockSpec((_ROW_TILE, D_MODEL), lambda i: (i, 0)),
        out_shape=jax.ShapeDtypeStruct((VOCAB, D_MODEL), jnp.float32),
    )(tableT, W, b.reshape(1, D_MODEL))


_NBUF = 5


def _make_gather(n_chunks):
    total = _NW * n_chunks * _CHUNK
    ngroups = n_chunks // _NBUF
    assert ngroups * _NBUF == n_chunks
    mesh = plsc.VectorSubcoreMesh(core_axis_name="c", subcore_axis_name="s")
    scratch = [pltpu.VMEM((n_chunks, _CHUNK), jnp.int32)]
    scratch += [pltpu.VMEM((_CHUNK, D_MODEL), jnp.float32) for _ in range(_NBUF)]
    scratch += [pltpu.SemaphoreType.DMA for _ in range(2 * _NBUF)]

    @functools.partial(
        pl.kernel,
        mesh=mesh,
        out_type=jax.ShapeDtypeStruct((total, D_MODEL), jnp.float32),
        scratch_types=scratch,
    )
    def gather_k(idx_hbm, p_hbm, out_hbm, idx_v, *bufs_sems):
        bufs = bufs_sems[:_NBUF]
        gsems = bufs_sems[_NBUF : 2 * _NBUF]
        wsems = bufs_sems[2 * _NBUF :]
        wid = lax.axis_index("s") * _NC + lax.axis_index("c")
        pltpu.sync_copy(idx_hbm.at[wid], idx_v)
        base = wid * (n_chunks * _CHUNK)

        def gath(j, bi):
            return pltpu.make_async_copy(p_hbm.at[idx_v.at[j]], bufs[bi], gsems[bi])

        def wb(j, bi):
            return pltpu.make_async_copy(
                bufs[bi], out_hbm.at[pl.ds(base + j * _CHUNK, _CHUNK)], wsems[bi]
            )

        for k in range(3):
            gath(k, k).start()

        def group(g, carry):
            j0 = g * _NBUF
            for bi in range(_NBUF):
                j = j0 + bi
                gath(j, bi).wait()
                wb(j, bi).start()
                nb = (bi + 3) % _NBUF

                @pl.when(j >= 2)
                def _():
                    wb(j - 2, nb).wait()

                @pl.when(j + 3 < n_chunks)
                def _():
                    gath(j + 3, nb).start()

            return carry

        lax.fori_loop(0, ngroups, group, 0)
        for j in (n_chunks - 2, n_chunks - 1):
            wb(j, j % _NBUF).wait()

    return gather_k


def kernel(morganSMILES, table, W, b):
    B, L = morganSMILES.shape
    total = B * L
    n_chunks = total // (_NW * _CHUNK)
    P = _project_table(table.T, W, b)
    idx = morganSMILES.T.reshape(_NW, n_chunks, _CHUNK).astype(jnp.int32)
    out = _make_gather(n_chunks)(idx, P)
    return out.reshape(L, B, D_MODEL).transpose(1, 0, 2)


# speedup vs baseline: 28.1064x; 1.0055x over previous
import functools

import jax
import jax.numpy as jnp
from jax import lax
from jax.experimental import pallas as pl
from jax.experimental.pallas import tpu as pltpu
from jax.experimental.pallas import tpu_sc as plsc

VOCAB = 100000
EMBED_DIM = 300
D_MODEL = 128

_ROW_TILE = 8192

_NC = 2
_NS = 16
_NW = _NC * _NS
_CHUNK = 128


def _proj_body(tT_ref, w_ref, b_ref, o_ref):
    o_ref[...] = (
        lax.dot_general(
            tT_ref[...],
            w_ref[...],
            dimension_numbers=(((0,), (0,)), ((), ())),
            preferred_element_type=jnp.float32,
        )
        + b_ref[...]
    )


def _project_table(tableT, W, b):
    grid = (VOCAB + _ROW_TILE - 1) // _ROW_TILE
    return pl.pallas_call(
        _proj_body,
        grid=(grid,),
        in_specs=[
            pl.BlockSpec((EMBED_DIM, _ROW_TILE), lambda i: (0, i)),
            pl.BlockSpec((EMBED_DIM, D_MODEL), lambda i: (0, 0)),
            pl.BlockSpec((1, D_MODEL), lambda i: (0, 0)),
        ],
        out_specs=pl.BlockSpec((_ROW_TILE, D_MODEL), lambda i: (i, 0)),
        out_shape=jax.ShapeDtypeStruct((VOCAB, D_MODEL), jnp.float32),
    )(tableT, W, b.reshape(1, D_MODEL))


_NBUF = 5


def _make_gather(n_chunks):
    total = _NW * n_chunks * _CHUNK
    ngroups = n_chunks // _NBUF
    assert ngroups * _NBUF == n_chunks
    mesh = plsc.VectorSubcoreMesh(core_axis_name="c", subcore_axis_name="s")
    scratch = [pltpu.VMEM((n_chunks, _CHUNK), jnp.int32)]
    scratch += [pltpu.VMEM((_CHUNK, D_MODEL), jnp.float32) for _ in range(_NBUF)]
    scratch += [pltpu.SemaphoreType.DMA for _ in range(2 * _NBUF)]

    @functools.partial(
        pl.kernel,
        mesh=mesh,
        out_type=jax.ShapeDtypeStruct((total, D_MODEL), jnp.float32),
        scratch_types=scratch,
    )
    def gather_k(idx_hbm, p_hbm, out_hbm, idx_v, *bufs_sems):
        bufs = bufs_sems[:_NBUF]
        gsems = bufs_sems[_NBUF : 2 * _NBUF]
        wsems = bufs_sems[2 * _NBUF :]
        wid = lax.axis_index("s") * _NC + lax.axis_index("c")
        pltpu.sync_copy(idx_hbm.at[wid], idx_v)
        base = wid * (n_chunks * _CHUNK)

        def gath(j, bi):
            return pltpu.make_async_copy(p_hbm.at[idx_v.at[j]], bufs[bi], gsems[bi])

        def wb(j, bi):
            return pltpu.make_async_copy(
                bufs[bi], out_hbm.at[pl.ds(base + j * _CHUNK, _CHUNK)], wsems[bi]
            )

        for bi in range(_NBUF):
            gath(bi, bi).start()

        def group(g, carry):
            j0 = g * _NBUF
            for bi in range(_NBUF):
                gath(j0 + bi, bi).wait()
                wb(j0 + bi, bi).start()

            @pl.when(g + 1 < ngroups)
            def _():
                for bi in range(_NBUF):
                    wb(j0 + bi, bi).wait()
                    gath(j0 + _NBUF + bi, bi).start()

            return carry

        lax.fori_loop(0, ngroups, group, 0)
        for bi in range(_NBUF):
            wb((ngroups - 1) * _NBUF + bi, bi).wait()

    return gather_k


def kernel(morganSMILES, table, W, b):
    B, L = morganSMILES.shape
    total = B * L
    n_chunks = total // (_NW * _CHUNK)
    P = _project_table(table.T, W, b)
    idx = morganSMILES.T.reshape(_NW, n_chunks, _CHUNK).astype(jnp.int32)
    out = _make_gather(n_chunks)(idx, P)
    return out.reshape(L, B, D_MODEL).transpose(1, 0, 2)


# speedup vs baseline: 28.4674x; 1.0128x over previous
import functools

import jax
import jax.numpy as jnp
from jax import lax
from jax.experimental import pallas as pl
from jax.experimental.pallas import tpu as pltpu
from jax.experimental.pallas import tpu_sc as plsc

VOCAB = 100000
EMBED_DIM = 300
D_MODEL = 128

_ROW_TILE = 12800

_NC = 2
_NS = 16
_NW = _NC * _NS
_CHUNK = 128


def _proj_body(tT_ref, w_ref, b_ref, o_ref):
    o_ref[...] = (
        lax.dot_general(
            tT_ref[...],
            w_ref[...],
            dimension_numbers=(((0,), (0,)), ((), ())),
            preferred_element_type=jnp.float32,
        )
        + b_ref[...]
    )


def _project_table(tableT, W, b):
    grid = (VOCAB + _ROW_TILE - 1) // _ROW_TILE
    return pl.pallas_call(
        _proj_body,
        grid=(grid,),
        in_specs=[
            pl.BlockSpec((EMBED_DIM, _ROW_TILE), lambda i: (0, i)),
            pl.BlockSpec((EMBED_DIM, D_MODEL), lambda i: (0, 0)),
            pl.BlockSpec((1, D_MODEL), lambda i: (0, 0)),
        ],
        out_specs=pl.BlockSpec((_ROW_TILE, D_MODEL), lambda i: (i, 0)),
        out_shape=jax.ShapeDtypeStruct((VOCAB, D_MODEL), jnp.float32),
    )(tableT, W, b.reshape(1, D_MODEL))


_NBUF = 5


def _make_gather(n_chunks):
    total = _NW * n_chunks * _CHUNK
    ngroups = n_chunks // _NBUF
    assert ngroups * _NBUF == n_chunks
    mesh = plsc.VectorSubcoreMesh(core_axis_name="c", subcore_axis_name="s")
    scratch = [pltpu.VMEM((n_chunks, _CHUNK), jnp.int32)]
    scratch += [pltpu.VMEM((_CHUNK, D_MODEL), jnp.float32) for _ in range(_NBUF)]
    scratch += [pltpu.SemaphoreType.DMA for _ in range(2 * _NBUF)]

    @functools.partial(
        pl.kernel,
        mesh=mesh,
        out_type=jax.ShapeDtypeStruct((total, D_MODEL), jnp.float32),
        scratch_types=scratch,
    )
    def gather_k(idx_hbm, p_hbm, out_hbm, idx_v, *bufs_sems):
        bufs = bufs_sems[:_NBUF]
        gsems = bufs_sems[_NBUF : 2 * _NBUF]
        wsems = bufs_sems[2 * _NBUF :]
        wid = lax.axis_index("s") * _NC + lax.axis_index("c")
        pltpu.sync_copy(idx_hbm.at[wid], idx_v)
        base = wid * (n_chunks * _CHUNK)

        def gath(j, bi):
            return pltpu.make_async_copy(p_hbm.at[idx_v.at[j]], bufs[bi], gsems[bi])

        def wb(j, bi):
            return pltpu.make_async_copy(
                bufs[bi], out_hbm.at[pl.ds(base + j * _CHUNK, _CHUNK)], wsems[bi]
            )

        for bi in range(_NBUF):
            gath(bi, bi).start()

        def group(g, carry):
            j0 = g * _NBUF
            for bi in range(_NBUF):
                gath(j0 + bi, bi).wait()
                wb(j0 + bi, bi).start()

            @pl.when(g + 1 < ngroups)
            def _():
                for bi in range(_NBUF):
                    wb(j0 + bi, bi).wait()
                    gath(j0 + _NBUF + bi, bi).start()

            return carry

        lax.fori_loop(0, ngroups, group, 0)
        for bi in range(_NBUF):
            wb((ngroups - 1) * _NBUF + bi, bi).wait()

    return gather_k


def kernel(morganSMILES, table, W, b):
    B, L = morganSMILES.shape
    total = B * L
    n_chunks = total // (_NW * _CHUNK)
    P = _project_table(table.T, W, b)
    idx = morganSMILES.T.reshape(_NW, n_chunks, _CHUNK).astype(jnp.int32)
    out = _make_gather(n_chunks)(idx, P)
    return out.reshape(L, B, D_MODEL).transpose(1, 0, 2)


# speedup vs baseline: 28.6637x; 1.0069x over previous
import functools

import jax
import jax.numpy as jnp
from jax import lax
from jax.experimental import pallas as pl
from jax.experimental.pallas import tpu as pltpu
from jax.experimental.pallas import tpu_sc as plsc

VOCAB = 100000
EMBED_DIM = 300
D_MODEL = 128

_ROW_TILE = 12800

_NC = 2
_NS = 16
_NW = _NC * _NS
_CHUNK = 64


def _proj_body(tT_ref, w_ref, b_ref, o_ref):
    o_ref[...] = (
        lax.dot_general(
            tT_ref[...],
            w_ref[...],
            dimension_numbers=(((0,), (0,)), ((), ())),
            preferred_element_type=jnp.float32,
        )
        + b_ref[...]
    )


def _project_table(tableT, W, b):
    grid = (VOCAB + _ROW_TILE - 1) // _ROW_TILE
    return pl.pallas_call(
        _proj_body,
        grid=(grid,),
        in_specs=[
            pl.BlockSpec((EMBED_DIM, _ROW_TILE), lambda i: (0, i)),
            pl.BlockSpec((EMBED_DIM, D_MODEL), lambda i: (0, 0)),
            pl.BlockSpec((1, D_MODEL), lambda i: (0, 0)),
        ],
        out_specs=pl.BlockSpec((_ROW_TILE, D_MODEL), lambda i: (i, 0)),
        out_shape=jax.ShapeDtypeStruct((VOCAB, D_MODEL), jnp.float32),
    )(tableT, W, b.reshape(1, D_MODEL))


_NBUF = 10


def _make_gather(n_chunks):
    total = _NW * n_chunks * _CHUNK
    ngroups = n_chunks // _NBUF
    assert ngroups * _NBUF == n_chunks
    mesh = plsc.VectorSubcoreMesh(core_axis_name="c", subcore_axis_name="s")
    scratch = [pltpu.VMEM((n_chunks, _CHUNK), jnp.int32)]
    scratch += [pltpu.VMEM((_CHUNK, D_MODEL), jnp.float32) for _ in range(_NBUF)]
    scratch += [pltpu.SemaphoreType.DMA for _ in range(2 * _NBUF)]

    @functools.partial(
        pl.kernel,
        mesh=mesh,
        out_type=jax.ShapeDtypeStruct((total, D_MODEL), jnp.float32),
        scratch_types=scratch,
    )
    def gather_k(idx_hbm, p_hbm, out_hbm, idx_v, *bufs_sems):
        bufs = bufs_sems[:_NBUF]
        gsems = bufs_sems[_NBUF : 2 * _NBUF]
        wsems = bufs_sems[2 * _NBUF :]
        wid = lax.axis_index("s") * _NC + lax.axis_index("c")
        pltpu.sync_copy(idx_hbm.at[wid], idx_v)
        base = wid * (n_chunks * _CHUNK)

        def gath(j, bi):
            return pltpu.make_async_copy(p_hbm.at[idx_v.at[j]], bufs[bi], gsems[bi])

        def wb(j, bi):
            return pltpu.make_async_copy(
                bufs[bi], out_hbm.at[pl.ds(base + j * _CHUNK, _CHUNK)], wsems[bi]
            )

        for bi in range(_NBUF):
            gath(bi, bi).start()

        def group(g, carry):
            j0 = g * _NBUF
            for bi in range(_NBUF):
                gath(j0 + bi, bi).wait()
                wb(j0 + bi, bi).start()

            @pl.when(g + 1 < ngroups)
            def _():
                for bi in range(_NBUF):
                    wb(j0 + bi, bi).wait()
                    gath(j0 + _NBUF + bi, bi).start()

            return carry

        lax.fori_loop(0, ngroups, group, 0)
        for bi in range(_NBUF):
            wb((ngroups - 1) * _NBUF + bi, bi).wait()

    return gather_k


def kernel(morganSMILES, table, W, b):
    B, L = morganSMILES.shape
    total = B * L
    n_chunks = total // (_NW * _CHUNK)
    P = _project_table(table.T, W, b)
    idx = morganSMILES.T.reshape(_NW, n_chunks, _CHUNK).astype(jnp.int32)
    out = _make_gather(n_chunks)(idx, P)
    return out.reshape(L, B, D_MODEL).transpose(1, 0, 2)
